# Initial kernel scaffold; baseline (speedup 1.0000x reference)
import jax
import jax.numpy as jnp
from jax.experimental import pallas as pl


def kernel(x, edge_values, W, b, edge_index):
    raise NotImplementedError("write your pallas kernel here")


# speedup vs baseline: 11.1693x; 11.1693x over previous
import functools

import jax
import jax.numpy as jnp
from jax import lax
from jax.experimental import pallas as pl
from jax.experimental.pallas import tpu as pltpu
from jax.experimental.pallas import tpu_sc as plsc

N_NODES = 10000
D_IN = 128
D_OUT = 128
NC = 2
NS = 16
LANES = 16
CHUNK = 128
N_PAD = 10240
ROWS_PER_TILE = N_PAD // NS
MM_BLOCK = 1000


def _matmul_body(x_ref, w_ref, b_ref, h_ref):
    wsum = w_ref[0] + w_ref[1] + w_ref[2] + w_ref[3]
    bsum = jnp.sum(b_ref[...], axis=0, keepdims=True)
    h_ref[...] = (
        jnp.dot(x_ref[...], wsum, preferred_element_type=jnp.float32) + bsum
    )


def _combine_body(p_ref, o_ref):
    o_ref[...] = jnp.maximum(p_ref[0] + p_ref[1], 0.0)


def _make_sc_kernel(cpw):
    mesh = plsc.VectorSubcoreMesh(core_axis_name="c", subcore_axis_name="s")

    @functools.partial(
        pl.kernel,
        mesh=mesh,
        out_type=jax.ShapeDtypeStruct((NC, N_PAD, D_OUT), jnp.float32),
        scratch_types=[
            pltpu.VMEM((cpw, CHUNK), jnp.int32),
            pltpu.VMEM((CHUNK, D_OUT), jnp.float32),
            pltpu.VMEM((CHUNK, D_OUT), jnp.float32),
            pltpu.VMEM((1, CHUNK), jnp.int32),
            pltpu.VMEM((1, CHUNK), jnp.int32),
            pltpu.VMEM((1, CHUNK), jnp.float32),
            pltpu.VMEM((1, CHUNK), jnp.float32),
            pltpu.VMEM_SHARED((N_PAD, D_OUT), jnp.float32),
            pltpu.SemaphoreType.DMA,
            pltpu.SemaphoreType.DMA,
            pltpu.SemaphoreType.DMA,
            pltpu.SemaphoreType.DMA,
            pltpu.SemaphoreType.DMA,
            pltpu.SemaphoreType.DMA,
        ],
    )
    def sc(h_hbm, cols_hbm, rows_hbm, ev_hbm, zeros_hbm, out_hbm,
           cols_v, buf0, buf1, rb0, rb1, eb0, eb1, acc,
           sg0, sg1, sr0, sr1, se0, se1):
        cid = lax.axis_index("c")
        sid = lax.axis_index("s")
        wid = cid * NS + sid
        base = wid * cpw

        pltpu.sync_copy(cols_hbm.at[pl.ds(base, cpw)], cols_v)
        pltpu.sync_copy(
            zeros_hbm.at[pl.ds(sid * ROWS_PER_TILE, ROWS_PER_TILE)],
            acc.at[pl.ds(sid * ROWS_PER_TILE, ROWS_PER_TILE)],
        )
        plsc.subcore_barrier()

        def scale_chunk(eb, buf):
            def group(g, carry):
                ev16 = eb[0, pl.ds(g * LANES, LANES)]
                for j in range(LANES):
                    idxj = jnp.full((LANES,), j, jnp.int32)
                    sj = ev16.at[idxj].get(mode="promise_in_bounds")
                    e = g * LANES + j
                    for s in range(D_OUT // LANES):
                        sl = (e, pl.ds(s * LANES, LANES))
                        buf[sl] = buf[sl] * sj
                return carry
            lax.fori_loop(0, CHUNK // LANES, group, 0)

        bufs = ((buf0, sg0, rb0, sr0, eb0, se0),
                (buf1, sg1, rb1, sr1, eb1, se1))
        for p, (buf, sg, rb, sr, eb, se) in enumerate(bufs):
            pltpu.async_copy(rows_hbm.at[pl.ds(base + p, 1)], rb, sr)
            pltpu.async_copy(ev_hbm.at[pl.ds(base + p, 1)], eb, se)
            pltpu.async_copy(h_hbm.at[cols_v.at[p]], buf, sg)

        def pair(c2, carry):
            for p, (buf, sg, rb, sr, eb, se) in enumerate(bufs):
                c = c2 + p
                pltpu.make_async_copy(
                    rows_hbm.at[pl.ds(base + c, 1)], rb, sr).wait()
                pltpu.make_async_copy(
                    ev_hbm.at[pl.ds(base + c, 1)], eb, se).wait()
                pltpu.make_async_copy(h_hbm.at[cols_v.at[c]], buf, sg).wait()
                scale_chunk(eb, buf)
                pltpu.sync_copy(buf, acc.at[rb.at[0]], add=True)

                @pl.when(c + 2 < cpw)
                def _():
                    pltpu.async_copy(rows_hbm.at[pl.ds(base + c + 2, 1)],
                                     rb, sr)
                    pltpu.async_copy(ev_hbm.at[pl.ds(base + c + 2, 1)],
                                     eb, se)
                    pltpu.async_copy(h_hbm.at[cols_v.at[c + 2]], buf, sg)
            return carry

        lax.fori_loop(0, cpw // 2, lambda i, cr: pair(i * 2, cr), 0)
        plsc.subcore_barrier()

        pltpu.sync_copy(
            acc.at[pl.ds(sid * ROWS_PER_TILE, ROWS_PER_TILE)],
            out_hbm.at[cid, pl.ds(sid * ROWS_PER_TILE, ROWS_PER_TILE)],
        )

    return sc


def kernel(x, edge_values, W, b, edge_index):
    n = x.shape[0]
    e = edge_values.shape[0]
    h = pl.pallas_call(
        _matmul_body,
        grid=(n // MM_BLOCK,),
        in_specs=[
            pl.BlockSpec((MM_BLOCK, D_IN), lambda i: (i, 0)),
            pl.BlockSpec((4, D_IN, D_OUT), lambda i: (0, 0, 0)),
            pl.BlockSpec((4, D_OUT), lambda i: (0, 0)),
        ],
        out_specs=pl.BlockSpec((MM_BLOCK, D_OUT), lambda i: (i, 0)),
        out_shape=jax.ShapeDtypeStruct((n, D_OUT), jnp.float32),
    )(x, W, b)

    chunks_per_worker = -(-e // (NC * NS * CHUNK))
    chunks_per_worker += chunks_per_worker % 2
    ep = NC * NS * chunks_per_worker * CHUNK
    pad = ep - e
    cols = jnp.concatenate(
        [edge_index[1], jnp.zeros((pad,), jnp.int32)]).reshape(-1, CHUNK)
    rows = jnp.concatenate(
        [edge_index[0], jnp.zeros((pad,), jnp.int32)]).reshape(-1, CHUNK)
    ev = jnp.concatenate(
        [edge_values, jnp.zeros((pad,), jnp.float32)]).reshape(-1, CHUNK)
    zeros = jnp.zeros((N_PAD, D_OUT), jnp.float32)

    partials = _make_sc_kernel(chunks_per_worker)(h, cols, rows, ev, zeros)

    out = pl.pallas_call(
        _combine_body,
        grid=(n // MM_BLOCK,),
        in_specs=[pl.BlockSpec((NC, MM_BLOCK, D_OUT), lambda i: (0, i, 0))],
        out_specs=pl.BlockSpec((MM_BLOCK, D_OUT), lambda i: (i, 0)),
        out_shape=jax.ShapeDtypeStruct((n, D_OUT), jnp.float32),
    )(partials)
    return out
